# Initial kernel scaffold; baseline (speedup 1.0000x reference)
import jax
import jax.numpy as jnp
from jax.experimental import pallas as pl


def kernel(sample_xyz, rel_xyz, fea, knn_idx, W1, b1, conv_dw):
    raise NotImplementedError("write your pallas kernel here")


# speedup vs baseline: 4.2276x; 4.2276x over previous
import functools

import jax
import jax.numpy as jnp
from jax import lax
from jax.experimental import pallas as pl
from jax.experimental.pallas import tpu as pltpu
from jax.experimental.pallas import tpu_sc as plsc

N = 10000
K = 32
C = 128
NP = 10240
NW = 32
PPW = NP // NW
CPC = 4
NCH = PPW // CPC
TN = 2048
KS3 = 125


def _sc_gather_sum(feaT, idx):
    mesh = plsc.VectorSubcoreMesh(
        core_axis_name="c", subcore_axis_name="s", num_cores=2, num_subcores=16
    )

    @functools.partial(
        pl.kernel,
        out_type=jax.ShapeDtypeStruct((NP, C), jnp.float32),
        mesh=mesh,
        scratch_types=[
            pltpu.VMEM((NCH, 128), jnp.int32),
            pltpu.VMEM((128, C), jnp.float32),
            pltpu.VMEM((128, C), jnp.float32),
            pltpu.VMEM((PPW, C), jnp.float32),
            pltpu.SemaphoreType.DMA,
            pltpu.SemaphoreType.DMA,
        ],
    )
    def sc_kernel(feaT_hbm, idx_hbm, out_hbm, idx_v, rows0, rows1, acc_v, sem0, sem1):
        wid = lax.axis_index("s") * 2 + lax.axis_index("c")
        pltpu.sync_copy(idx_hbm.at[wid], idx_v)

        rows = (rows0, rows1)
        sems = (sem0, sem1)

        pltpu.async_copy(feaT_hbm.at[idx_v.at[0]], rows0, sem0)
        pltpu.async_copy(feaT_hbm.at[idx_v.at[1]], rows1, sem1)

        def two_chunks(i, carry):
            c0 = i * 2
            for b in range(2):
                c = c0 + b
                rb, sb = rows[b], sems[b]
                pltpu.make_async_copy(feaT_hbm.at[idx_v.at[c]], rb, sb).wait()
                for p in range(CPC):
                    r0 = p * K
                    for v in range(C // 16):
                        sl = pl.ds(v * 16, 16)
                        s = rb[r0, sl]
                        for j in range(1, K):
                            s = s + rb[r0 + j, sl]
                        acc_v[c * CPC + p, sl] = s

                @pl.when(c + 2 < NCH)
                def _():
                    pltpu.async_copy(feaT_hbm.at[idx_v.at[c + 2]], rb, sb)

            return carry

        lax.fori_loop(0, NCH // 2, two_chunks, 0)
        pltpu.sync_copy(acc_v, out_hbm.at[pl.ds(wid * PPW, PPW)])

    return sc_kernel(feaT, idx)


def _tc_body(s_ref, fea_ref, rel_ref, smp_ref, w1f_ref, w1x_ref, b1_ref, dw_ref, out_ref):
    w1f = w1f_ref[:]
    z = lax.dot_general(
        w1f, s_ref[:], (((1,), (1,)), ((), ())), preferred_element_type=jnp.float32
    )
    z = z - 30.0 * jnp.dot(w1f, fea_ref[:], preferred_element_type=jnp.float32)
    rel_s = jnp.sum(rel_ref[:], axis=1)
    z = z + jnp.dot(w1x_ref[:], rel_s, preferred_element_type=jnp.float32)
    z = z + 32.0 * b1_ref[:]

    smp = jnp.clip(smp_ref[:], -0.99999, 0.99999) * (5.0 / 2.0)
    coord = smp.astype(jnp.int32) + 2
    pos = coord[2:3, :] * 25 + coord[1:2, :] * 5 + coord[0:1, :]
    onehot = (
        lax.broadcasted_iota(jnp.int32, (KS3, TN), 0) == pos
    ).astype(jnp.float32)
    kern = jnp.dot(dw_ref[:], onehot, preferred_element_type=jnp.float32)
    out_ref[:] = kern * z


def _tc_combine(S, fea2d, rel3, smpT, W1f, W1x, b1c, dw):
    return pl.pallas_call(
        _tc_body,
        grid=((N + TN - 1) // TN,),
        in_specs=[
            pl.BlockSpec((TN, C), lambda i: (i, 0)),
            pl.BlockSpec((C, TN), lambda i: (0, i)),
            pl.BlockSpec((3, K, TN), lambda i: (0, 0, i)),
            pl.BlockSpec((3, TN), lambda i: (0, i)),
            pl.BlockSpec((C, C), lambda i: (0, 0)),
            pl.BlockSpec((C, 3), lambda i: (0, 0)),
            pl.BlockSpec((C, 1), lambda i: (0, 0)),
            pl.BlockSpec((C, KS3), lambda i: (0, 0)),
        ],
        out_specs=pl.BlockSpec((C, TN), lambda i: (0, i)),
        out_shape=jax.ShapeDtypeStruct((C, N), jnp.float32),
    )(S, fea2d, rel3, smpT, W1f, W1x, b1c, dw)


def kernel(sample_xyz, rel_xyz, fea, knn_idx, W1, b1, conv_dw):
    fea2d = fea[0]
    feaT = jnp.pad(fea2d.T, ((0, NP - N), (0, 0)))
    idx = knn_idx[0, :, 1:]
    idx = jnp.concatenate(
        [idx, jnp.full((N, 1), N, dtype=jnp.int32)], axis=1
    )
    idx = jnp.pad(idx, ((0, NP - N), (0, 0)), constant_values=N)
    idx = idx.reshape(NW, NCH, 128)

    S = _sc_gather_sum(feaT, idx)

    out2d = _tc_combine(
        S,
        fea2d,
        rel_xyz[0],
        sample_xyz[0].T,
        W1[:, :C],
        W1[:, C:],
        b1.reshape(C, 1),
        conv_dw[0].reshape(C, KS3),
    )
    return out2d[None]
